# baseline (speedup 1.0000x reference)
import jax
import jax.numpy as jnp
from jax import lax
from jax.experimental import pallas as pl
from jax.experimental.pallas import tpu as pltpu
from jax.experimental.pallas import tpu_sc as plsc

_N_FIELDS = 26
_VOCAB = 100000
_EMB = 64
_BATCH = 4096
_NC, _NS, _L = 2, 16, 16
_NW = _NC * _NS
_NTASK = _N_FIELDS * _EMB
_TPW = _NTASK // _NW
_W0 = 33408
_W1 = 33408
_W2 = _VOCAB - _W0 - _W1
_WS = (_W0, _W1, _W2)
_OFFS = (0, _W0, _W0 + _W1)
_NG = _BATCH // _L


def _tile_body(xT_hbm, tt_hbm, outT_hbm, tvA, tvB, tvC, xf, ov0, ov1,
               tsemA, tsemB, tsemC, ssem0, ssem1):
    wid = lax.axis_index("s") * _NC + lax.axis_index("c")
    base = wid * _TPW
    i_first = base // _EMB
    tvs = (tvA, tvB, tvC)
    tsems = (tsemA, tsemB, tsemC)

    pltpu.sync_copy(xT_hbm.at[i_first], xf.at[pl.ds(0, _BATCH)])
    pltpu.sync_copy(xT_hbm.at[(base + _TPW - 1) // _EMB],
                    xf.at[pl.ds(_BATCH, _BATCH)])

    def stream_part(t, p):
        i = t // _EMB
        c = t % _EMB
        return pltpu.make_async_copy(
            tt_hbm.at[i, c, pl.ds(_OFFS[p], _WS[p])], tvs[p], tsems[p])

    def gather_part(t, ov, p):
        xoff = (t // _EMB - i_first) * _BATCH

        def grp(g, _):
            for u in range(16):
                off = (g * 16 + u) * _L
                idx = xf[pl.ds(xoff + off, _L)]
                if p == 0:
                    m = idx < _W0
                    v = plsc.load_gather(tvA, [idx], mask=m)
                    ov[pl.ds(off, _L)] = v
                else:
                    lo = _OFFS[p]
                    m = idx >= lo if p == 2 else (idx >= lo) & (idx < lo + _WS[p])
                    v = plsc.load_gather(tvs[p], [idx - lo], mask=m)
                    ov[pl.ds(off, _L)] = jnp.where(m, v, ov[pl.ds(off, _L)])
            return 0

        lax.fori_loop(0, _NG // 16, grp, 0)

    def do_task(t, ov, ssem, nxt, nxt_guard):
        def refill(p):
            if nxt_guard is None:
                stream_part(nxt, p).start()
            else:
                @pl.when(nxt_guard)
                def _():
                    stream_part(nxt, p).start()

        for p in range(3):
            stream_part(t, p).wait()
            gather_part(t, ov, p)
            refill(p)
        pltpu.make_async_copy(ov, outT_hbm.at[t], ssem).start()

    for p in range(3):
        stream_part(base, p).start()

    def body(j, _):
        t0 = base + 2 * j

        @pl.when(j >= 1)
        def _():
            pltpu.make_async_copy(ov0, outT_hbm.at[t0], ssem0).wait()
        do_task(t0, ov0, ssem0, t0 + 1, None)

        @pl.when(j >= 1)
        def _():
            pltpu.make_async_copy(ov1, outT_hbm.at[t0], ssem1).wait()
        do_task(t0 + 1, ov1, ssem1, t0 + 2, j <= _TPW // 2 - 2)
        return 0

    lax.fori_loop(0, _TPW // 2, body, 0)
    pltpu.make_async_copy(ov0, outT_hbm.at[base], ssem0).wait()
    pltpu.make_async_copy(ov1, outT_hbm.at[base], ssem1).wait()


def kernel(x, tables):
    xT = jnp.transpose(x)
    tt = jnp.transpose(tables, (0, 2, 1))
    mesh = plsc.VectorSubcoreMesh(core_axis_name="c", subcore_axis_name="s")
    outT = pl.kernel(
        _tile_body,
        out_type=jax.ShapeDtypeStruct((_NTASK, _BATCH), jnp.float32),
        mesh=mesh,
        compiler_params=pltpu.CompilerParams(needs_layout_passes=False),
        scratch_types=[
            pltpu.VMEM((_W0,), jnp.float32),
            pltpu.VMEM((_W1,), jnp.float32),
            pltpu.VMEM((_W2,), jnp.float32),
            pltpu.VMEM((2 * _BATCH,), jnp.int32),
            pltpu.VMEM((_BATCH,), jnp.float32),
            pltpu.VMEM((_BATCH,), jnp.float32),
            pltpu.SemaphoreType.DMA,
            pltpu.SemaphoreType.DMA,
            pltpu.SemaphoreType.DMA,
            pltpu.SemaphoreType.DMA,
            pltpu.SemaphoreType.DMA,
        ],
    )(xT, tt)
    return jnp.transpose(outT)


# speedup vs baseline: 1.4070x; 1.4070x over previous
import jax
import jax.numpy as jnp
from jax import lax
from jax.experimental import pallas as pl
from jax.experimental.pallas import tpu as pltpu
from jax.experimental.pallas import tpu_sc as plsc

_N_FIELDS = 26
_VOCAB = 100000
_EMB = 64
_BATCH = 4096
_NC, _NS, _L = 2, 16, 16
_NW = _NC * _NS
_NTASK = _N_FIELDS * _EMB
_TPW = _NTASK // _NW
_W0 = 50048
_W1 = _VOCAB - _W0
_NG = _BATCH // _L


def _tile_body(xT_hbm, tt_hbm, outT_hbm, tvA, tvB, xf, ov0, ov1,
               tsemA, tsemB, ssem0, ssem1):
    wid = lax.axis_index("s") * _NC + lax.axis_index("c")
    base = wid * _TPW
    i_first = base // _EMB

    pltpu.sync_copy(xT_hbm.at[i_first], xf.at[pl.ds(0, _BATCH)])
    pltpu.sync_copy(xT_hbm.at[(base + _TPW - 1) // _EMB],
                    xf.at[pl.ds(_BATCH, _BATCH)])

    def stream_half(t, half):
        i = t // _EMB
        c = t % _EMB
        if half == 0:
            return pltpu.make_async_copy(
                tt_hbm.at[i, c, pl.ds(0, _W0)], tvA, tsemA)
        return pltpu.make_async_copy(
            tt_hbm.at[i, c, pl.ds(_W0, _W1)], tvB, tsemB)

    def gather_half(t, ov, half):
        xoff = (t // _EMB - i_first) * _BATCH

        def grp(g, _):
            for u in range(16):
                off = (g * 16 + u) * _L
                idx = xf[pl.ds(xoff + off, _L)]
                if half == 0:
                    m = idx < _W0
                    v = plsc.load_gather(tvA, [idx], mask=m)
                    ov[pl.ds(off, _L)] = v
                else:
                    m = idx >= _W0
                    v = plsc.load_gather(tvB, [idx - _W0], mask=m)
                    ov[pl.ds(off, _L)] = jnp.where(m, v, ov[pl.ds(off, _L)])
            return 0

        lax.fori_loop(0, _NG // 16, grp, 0)

    def do_task(t, ov, ssem, nxt, nxt_guard):
        def refill(half):
            if nxt_guard is None:
                stream_half(nxt, half).start()
            else:
                @pl.when(nxt_guard)
                def _():
                    stream_half(nxt, half).start()

        stream_half(t, 0).wait()
        gather_half(t, ov, 0)
        refill(0)
        stream_half(t, 1).wait()
        gather_half(t, ov, 1)
        refill(1)
        pltpu.make_async_copy(ov, outT_hbm.at[t], ssem).start()

    stream_half(base, 0).start()
    stream_half(base, 1).start()

    def body(j, _):
        t0 = base + 2 * j

        @pl.when(j >= 1)
        def _():
            pltpu.make_async_copy(ov0, outT_hbm.at[t0], ssem0).wait()
        do_task(t0, ov0, ssem0, t0 + 1, None)

        @pl.when(j >= 1)
        def _():
            pltpu.make_async_copy(ov1, outT_hbm.at[t0], ssem1).wait()
        do_task(t0 + 1, ov1, ssem1, t0 + 2, j <= _TPW // 2 - 2)
        return 0

    lax.fori_loop(0, _TPW // 2, body, 0)
    pltpu.make_async_copy(ov0, outT_hbm.at[base], ssem0).wait()
    pltpu.make_async_copy(ov1, outT_hbm.at[base], ssem1).wait()


def kernel(x, tables):
    xT = jnp.transpose(x)
    tt = jnp.transpose(tables, (0, 2, 1))
    mesh = plsc.VectorSubcoreMesh(core_axis_name="c", subcore_axis_name="s")
    outT = pl.kernel(
        _tile_body,
        out_type=jax.ShapeDtypeStruct((_NTASK, _BATCH), jnp.float32),
        mesh=mesh,
        compiler_params=pltpu.CompilerParams(needs_layout_passes=False),
        scratch_types=[
            pltpu.VMEM((_W0,), jnp.float32),
            pltpu.VMEM((_W1,), jnp.float32),
            pltpu.VMEM((2 * _BATCH,), jnp.int32),
            pltpu.VMEM((_BATCH,), jnp.float32),
            pltpu.VMEM((_BATCH,), jnp.float32),
            pltpu.SemaphoreType.DMA,
            pltpu.SemaphoreType.DMA,
            pltpu.SemaphoreType.DMA,
            pltpu.SemaphoreType.DMA,
        ],
    )(xT, tt)
    return jnp.transpose(outT)
